# baseline (speedup 1.0000x reference)
import functools

import jax
import jax.numpy as jnp
from jax import lax
from jax.experimental import pallas as pl
from jax.experimental.pallas import tpu as pltpu
from jax.experimental.pallas import tpu_sc as plsc


def kernel(x, emb_table, pos_table):
    B, S = x.shape
    V, D = emb_table.shape
    scale = float(D) ** 0.5
    NVEC = D // 16

    info = plsc.get_sparse_core_info()
    NC, NS = info.num_cores, info.num_subcores
    NW = NC * NS
    seqs_per_w = B // NW

    C0 = 104
    C1 = S - C0

    NB = 4
    RU = 4

    pos = pos_table[:S]

    mesh = plsc.VectorSubcoreMesh(core_axis_name="c", subcore_axis_name="s")

    @functools.partial(
        pl.kernel,
        mesh=mesh,
        compiler_params=pltpu.CompilerParams(use_tc_tiling_on_sc=False),
        out_type=jax.ShapeDtypeStruct((B, S, D), jnp.float32),
        scratch_types=[
            pltpu.VMEM((NB, S), jnp.int32),
            pltpu.VMEM((NB, S, D), jnp.float32),
            pltpu.VMEM((S, D), jnp.float32),
            pltpu.SemaphoreType.DMA((NB,)),
            pltpu.SemaphoreType.DMA((NB,)),
        ],
    )
    def emb_kernel(x_hbm, tab_hbm, pos_hbm, out_hbm, idx_v, rows_v, pos_v,
                   gsem, wsem):
        wid = lax.axis_index("s") * NC + lax.axis_index("c")
        base = wid * seqs_per_w
        pltpu.sync_copy(pos_hbm, pos_v)

        def start_fetch(j, b):
            pltpu.sync_copy(x_hbm.at[base + j], idx_v.at[b])
            pltpu.async_copy(
                tab_hbm.at[idx_v.at[b, pl.ds(0, C0)]],
                rows_v.at[b, pl.ds(0, C0)],
                gsem.at[b],
            )
            pltpu.async_copy(
                tab_hbm.at[idx_v.at[b, pl.ds(C0, C1)]],
                rows_v.at[b, pl.ds(C0, C1)],
                gsem.at[b],
            )

        def wait_gather(b):
            pltpu.make_async_copy(
                tab_hbm.at[idx_v.at[b, pl.ds(0, C0)]],
                rows_v.at[b, pl.ds(0, C0)], gsem.at[b]).wait()
            pltpu.make_async_copy(
                tab_hbm.at[idx_v.at[b, pl.ds(C0, C1)]],
                rows_v.at[b, pl.ds(C0, C1)], gsem.at[b]).wait()

        def wait_wb(j, b):
            pltpu.make_async_copy(
                rows_v.at[b], out_hbm.at[base + j], wsem.at[b]).wait()

        def compute(b):
            def body(i, carry):
                r = i * RU
                for rr in range(RU):
                    for c in range(NVEC):
                        sl = pl.ds(c * 16, 16)
                        rows_v[b, r + rr, sl] = (
                            rows_v[b, r + rr, sl] * scale + pos_v[r + rr, sl]
                        )
                return carry

            lax.fori_loop(0, S // RU, body, 0, unroll=False)

        start_fetch(0, 0)
        start_fetch(1, 1)

        def seq_body(j, carry):
            b = j % NB
            f = j + 2
            fb = f % NB

            @pl.when(f < seqs_per_w)
            def _():
                @pl.when(f >= NB)
                def _():
                    wait_wb(f - NB, fb)
                start_fetch(f, fb)

            wait_gather(b)
            compute(b)
            pltpu.async_copy(rows_v.at[b], out_hbm.at[base + j], wsem.at[b])
            return carry

        lax.fori_loop(0, seqs_per_w, seq_body, 0, unroll=False)
        wait_wb(seqs_per_w - 2, (seqs_per_w - 2) % NB)
        wait_wb(seqs_per_w - 1, (seqs_per_w - 1) % NB)

    return emb_kernel(x, emb_table, pos)
